# baseline (speedup 1.0000x reference)
import functools

import jax
import jax.numpy as jnp
from jax import lax
from jax.experimental import pallas as pl
from jax.experimental.pallas import tpu as pltpu
from jax.experimental.pallas import tpu_sc as plsc

V = 8
S = 7
D = 64
B = 16384

NC = 2
NS = 16
L = 16
NW = NC * NS
BPW = B // NW
CHUNK = 128
NCH = BPW // CHUNK
GPC = CHUNK // L

_MESH = plsc.VectorSubcoreMesh(core_axis_name="c", subcore_axis_name="s")
_CP = pltpu.CompilerParams(use_tc_tiling_on_sc=True, needs_layout_passes=False)


def _worker_base():
    wid = lax.axis_index("s") * NC + lax.axis_index("c")
    return wid * BPW


def _zero_rows(buf, nrows, ncols):
    def body(r, _):
        for c in range(ncols // L):
            buf[r, pl.ds(c * L, L)] = jnp.zeros((L,), jnp.float32)
        return _
    lax.fori_loop(0, nrows, body, None)


def _gather_transpose_write(table, rows0, idx_row, idx_v, pbuf, gbuf, obuf,
                            sem, out_ref, base):
    copies = [pltpu.make_async_copy(table.at[idx_v.at[idx_row, j]],
                                    gbuf.at[pl.ds(j * CHUNK, CHUNK)], sem)
              for j in range(NCH)]
    for c in copies:
        c.start()
    for c in copies:
        c.wait()

    iota = lax.iota(jnp.int32, L)

    def tcol(i, _):
        row = i * L + iota
        pv = pbuf[pl.ds(i * L, L)]
        for d in range(D):
            v = plsc.load_gather(gbuf, [row, pv + d])
            obuf[d, pl.ds(i * L, L)] = v
        return _
    lax.fori_loop(0, BPW // L, tcol, None)

    for dg in range(D // 8):
        pltpu.sync_copy(obuf.at[pl.ds(dg * 8, 8)],
                        out_ref.at[pl.ds(rows0 + dg * 8, 8),
                                   pl.ds(base, BPW)])


def _k1_body(inT, t1, t2, t3, t4, out_ref, e56_out,
             in_v, idx_v, pbuf, gbuf, obuf, ebuf, sem):
    base = _worker_base()

    pltpu.sync_copy(inT.at[pl.ds(0, 6), :, pl.ds(base, BPW)], in_v)

    _zero_rows(obuf, D, BPW)
    for dg in range(D // 8):
        pltpu.sync_copy(obuf.at[pl.ds(dg * 8, 8)],
                        out_ref.at[pl.ds(dg * 8, 8), pl.ds(base, BPW)])
    _zero_rows(ebuf, 8, BPW)

    for j in range(NCH):
        def amax(g, _):
            off = j * CHUNK + g * L
            e = jnp.zeros((L,), jnp.int32)
            for s in range(S - 1):
                m = in_v[s, 0, pl.ds(off, L)]
                a = jnp.zeros((L,), jnp.int32)
                for v in range(1, V):
                    val = in_v[s, v, pl.ds(off, L)]
                    gt = val > m
                    m = jnp.where(gt, val, m)
                    a = jnp.where(gt, jnp.full((L,), v, jnp.int32), a)
                if s == 0:
                    pbuf[pl.ds(off, L)] = (a & 1) * D
                e = e + a * (V ** s)
                if s < 4:
                    idx_v[s, j, pl.ds(g * L, L)] = e >> 1
                else:
                    ebuf[s - 4, pl.ds(off, L)] = plsc.bitcast(e, jnp.float32)
            return _
        lax.fori_loop(0, GPC, amax, None)

    for t, tab in enumerate([t1, t2, t3, t4]):
        _gather_transpose_write(tab, (t + 1) * D, t, idx_v, pbuf, gbuf, obuf,
                                sem, out_ref, base)

    pltpu.sync_copy(ebuf, e56_out.at[:, pl.ds(base, BPW)])


def _k2_body(t5, t6, e56, out_ref, ebuf, idx_v, pbuf, gbuf, obuf, sem):
    base = _worker_base()
    pltpu.sync_copy(e56.at[:, pl.ds(base, BPW)], ebuf)

    for j in range(NCH):
        for g in range(GPC):
            off = j * CHUNK + g * L
            e5 = plsc.bitcast(ebuf[0, pl.ds(off, L)], jnp.int32)
            e6 = plsc.bitcast(ebuf[1, pl.ds(off, L)], jnp.int32)
            idx_v[0, j, pl.ds(g * L, L)] = e5 >> 1
            idx_v[1, j, pl.ds(g * L, L)] = e6 >> 1
            pbuf[pl.ds(off, L)] = (e5 & 1) * D

    _gather_transpose_write(t5, 5 * D, 0, idx_v, pbuf, gbuf, obuf, sem,
                            out_ref, base)
    _gather_transpose_write(t6, 6 * D, 1, idx_v, pbuf, gbuf, obuf, sem,
                            out_ref, base)


_k1 = functools.partial(
    pl.kernel,
    out_type=jax.ShapeDtypeStruct((8, B), jnp.float32),
    mesh=_MESH,
    compiler_params=_CP,
    scratch_types=[
        pltpu.VMEM((6, V, BPW), jnp.float32),
        pltpu.VMEM((4, NCH, CHUNK), jnp.int32),
        pltpu.VMEM((BPW,), jnp.int32),
        pltpu.VMEM((BPW, 2 * D), jnp.float32),
        pltpu.VMEM((D, BPW), jnp.float32),
        pltpu.VMEM((8, BPW), jnp.float32),
        pltpu.SemaphoreType.DMA,
    ],
)(_k1_body)

_k2 = functools.partial(
    pl.kernel,
    out_type=(),
    mesh=_MESH,
    compiler_params=_CP,
    scratch_types=[
        pltpu.VMEM((8, BPW), jnp.float32),
        pltpu.VMEM((2, NCH, CHUNK), jnp.int32),
        pltpu.VMEM((BPW,), jnp.int32),
        pltpu.VMEM((BPW, 2 * D), jnp.float32),
        pltpu.VMEM((D, BPW), jnp.float32),
        pltpu.SemaphoreType.DMA,
    ],
)(_k2_body)


@jax.jit
def _run(inputs, t1, t2, t3, t4, t5, t6):
    inT = jnp.transpose(inputs, (1, 2, 0))
    out_ref = jax.new_ref(jnp.zeros((S * D, B), jnp.float32))
    e56 = _k1(inT, t1.reshape(-1, 2 * D), t2.reshape(-1, 2 * D),
              t3.reshape(-1, 2 * D), t4.reshape(-1, 2 * D), out_ref)
    _k2(t5.reshape(-1, 2 * D), t6.reshape(-1, 2 * D), e56, out_ref)
    out = out_ref[...]
    return jnp.transpose(out, (1, 0)).reshape(B, S, D)


def kernel(inputs, table_1, table_2, table_3, table_4, table_5, table_6):
    return _run(inputs, table_1, table_2, table_3, table_4,
                table_5, table_6)
